# baseline (speedup 1.0000x reference)
import functools

import jax
import jax.numpy as jnp
from jax import lax
from jax.experimental import pallas as pl
from jax.experimental.pallas import tpu as pltpu
from jax.experimental.pallas import tpu_sc as plsc

N, C = 1024, 100000
BC = 2048
NB = (C + BC - 1) // BC
LAST = NB - 1

_NC, _NS, _L = 2, 16, 16
_NW = _NC * _NS
_BPW = N // _NW


def _sc_gather_body(cos_hbm, phi_hbm, tgt_hbm, cost_out, phit_out,
                    tgt_v, idx_v, cg_v, pg_v, sem):
    wid = lax.axis_index("s") * _NC + lax.axis_index("c")
    base = wid * _BPW
    pltpu.sync_copy(tgt_hbm.at[pl.ds(base, _BPW)], tgt_v)
    for j in range(_BPW // _L):
        t = tgt_v[pl.ds(j * _L, _L)]
        row = base + j * _L + lax.iota(jnp.int32, _L)
        idx_v[pl.ds(j * _L, _L)] = row * C + t
    pltpu.async_copy(cos_hbm.at[idx_v], cg_v, sem).wait()
    pltpu.async_copy(phi_hbm.at[idx_v], pg_v, sem).wait()
    pltpu.sync_copy(cg_v, cost_out.at[pl.ds(base, _BPW)])
    pltpu.sync_copy(pg_v, phit_out.at[pl.ds(base, _BPW)])


@functools.cache
def _sc_gather():
    return functools.partial(
        pl.kernel,
        out_type=(jax.ShapeDtypeStruct((N,), jnp.float32),
                  jax.ShapeDtypeStruct((N,), jnp.float32)),
        mesh=plsc.VectorSubcoreMesh(core_axis_name="c", subcore_axis_name="s",
                                    num_cores=_NC, num_subcores=_NS),
        scratch_types=[
            pltpu.VMEM((_BPW,), jnp.int32),
            pltpu.VMEM((_BPW,), jnp.int32),
            pltpu.VMEM((_BPW,), jnp.float32),
            pltpu.VMEM((_BPW,), jnp.float32),
            pltpu.SemaphoreType.DMA,
        ],
    )(_sc_gather_body)


def _tc_body(cos_ref, cost_ref, phit_ref, out_ref, m_ref, s_ref):
    k = pl.program_id(0)

    @pl.when(k == 0)
    def _init():
        m_ref[...] = jnp.full(m_ref.shape, -jnp.inf, jnp.float32)
        s_ref[...] = jnp.zeros(s_ref.shape, jnp.float32)

    def update(block):
        m = m_ref[...]
        bm = jnp.max(block, axis=1, keepdims=True)
        mn = jnp.maximum(m, bm)
        s_ref[...] = (s_ref[...] * jnp.exp(m - mn)
                      + jnp.sum(jnp.exp(block - mn), axis=1, keepdims=True))
        m_ref[...] = mn

    @pl.when(k < LAST)
    def _full():
        update(cos_ref[...])

    @pl.when(k == LAST)
    def _last():
        block = cos_ref[...]
        ids = lax.broadcasted_iota(jnp.int32, block.shape, 1) + k * BC
        update(jnp.where(ids < C, block, -jnp.inf))
        m = m_ref[...]
        s = s_ref[...]
        phit = phit_ref[...]
        cost = cost_ref[...]
        mf = jnp.maximum(m, phit)
        z = s * jnp.exp(m - mf) - jnp.exp(cost - mf) + jnp.exp(phit - mf)
        nll = mf + jnp.log(z) - phit
        out_ref[...] = (jnp.sum(nll) * (1.0 / N)).reshape(1, 1)


_tc_pass = pl.pallas_call(
    _tc_body,
    grid=(NB,),
    in_specs=[
        pl.BlockSpec((N, BC), lambda k: (0, k)),
        pl.BlockSpec((N, 1), lambda k: (0, 0)),
        pl.BlockSpec((N, 1), lambda k: (0, 0)),
    ],
    out_specs=pl.BlockSpec((1, 1), lambda k: (0, 0)),
    out_shape=jax.ShapeDtypeStruct((1, 1), jnp.float32),
    scratch_shapes=[
        pltpu.VMEM((N, 1), jnp.float32),
        pltpu.VMEM((N, 1), jnp.float32),
    ],
    compiler_params=pltpu.CompilerParams(
        dimension_semantics=("arbitrary",),
    ),
)


@jax.jit
def _impl(cos, phi, target):
    cos_t, phi_t = _sc_gather()(cos.reshape(-1), phi.reshape(-1), target)
    loss = _tc_pass(cos, cos_t.reshape(N, 1), phi_t.reshape(N, 1))
    return loss[0, 0]


def kernel(cos, phi, target):
    return _impl(cos, phi, target)


# speedup vs baseline: 2.1522x; 2.1522x over previous
import functools

import jax
import jax.numpy as jnp
from jax import lax
from jax.experimental import pallas as pl
from jax.experimental.pallas import tpu as pltpu
from jax.experimental.pallas import tpu_sc as plsc

N, C = 1024, 100000
BC = 2048
NB = (C + BC - 1) // BC
LAST = NB - 1

_NC, _NS, _L = 2, 16, 16
_NW = _NC * _NS
_BPW = N // _NW


def _sc_gather_body(cos_hbm, phi_hbm, tgt_hbm, cost_out, phit_out,
                    tgt_v, cwin, pwin, cg_v, pg_v, sem):
    wid = lax.axis_index("s") * _NC + lax.axis_index("c")
    base = wid * _BPW
    pltpu.sync_copy(tgt_hbm.at[pl.ds(base, _BPW)], tgt_v)
    copies = []
    for r in range(_BPW):
        if r % _L == 0:
            tgrp = tgt_v[pl.ds(r, _L)]
        t = tgrp[r % _L]
        c0 = pl.multiple_of(jnp.bitwise_and(t, -128), 128)
        r0 = (r // 8) * 8
        copies.append(pltpu.async_copy(
            cos_hbm.at[pl.ds(base + r0, 8), pl.ds(c0, 128)],
            cwin.at[pl.ds(r * 8, 8)], sem))
        copies.append(pltpu.async_copy(
            phi_hbm.at[pl.ds(base + r0, 8), pl.ds(c0, 128)],
            pwin.at[pl.ds(r * 8, 8)], sem))
    for cp in copies:
        cp.wait()
    for j in range(_BPW // _L):
        tv = tgt_v[pl.ds(j * _L, _L)]
        off = jnp.bitwise_and(tv, 127)
        r = j * _L + lax.iota(jnp.int32, _L)
        rows = r * 8 + jnp.bitwise_and(r, 7)
        cg_v[pl.ds(j * _L, _L)] = plsc.load_gather(cwin, [rows, off])
        pg_v[pl.ds(j * _L, _L)] = plsc.load_gather(pwin, [rows, off])
    pltpu.sync_copy(cg_v, cost_out.at[pl.ds(base, _BPW)])
    pltpu.sync_copy(pg_v, phit_out.at[pl.ds(base, _BPW)])


@functools.cache
def _sc_gather():
    return functools.partial(
        pl.kernel,
        out_type=(jax.ShapeDtypeStruct((N,), jnp.float32),
                  jax.ShapeDtypeStruct((N,), jnp.float32)),
        mesh=plsc.VectorSubcoreMesh(core_axis_name="c", subcore_axis_name="s",
                                    num_cores=_NC, num_subcores=_NS),
        scratch_types=[
            pltpu.VMEM((_BPW,), jnp.int32),
            pltpu.VMEM((_BPW * 8, 128), jnp.float32),
            pltpu.VMEM((_BPW * 8, 128), jnp.float32),
            pltpu.VMEM((_BPW,), jnp.float32),
            pltpu.VMEM((_BPW,), jnp.float32),
            pltpu.SemaphoreType.DMA,
        ],
        compiler_params=pltpu.CompilerParams(needs_layout_passes=False),
    )(_sc_gather_body)


def _tc_body(cos_ref, cost_ref, phit_ref, out_ref, m_ref, s_ref):
    k = pl.program_id(0)

    @pl.when(k == 0)
    def _init():
        m_ref[...] = jnp.full(m_ref.shape, -jnp.inf, jnp.float32)
        s_ref[...] = jnp.zeros(s_ref.shape, jnp.float32)

    def update(block):
        m = m_ref[...]
        bm = jnp.max(block, axis=1, keepdims=True)
        mn = jnp.maximum(m, bm)
        s_ref[...] = (s_ref[...] * jnp.exp(m - mn)
                      + jnp.sum(jnp.exp(block - mn), axis=1, keepdims=True))
        m_ref[...] = mn

    @pl.when(k < LAST)
    def _full():
        update(cos_ref[...])

    @pl.when(k == LAST)
    def _last():
        block = cos_ref[...]
        ids = lax.broadcasted_iota(jnp.int32, block.shape, 1) + k * BC
        update(jnp.where(ids < C, block, -jnp.inf))
        m = m_ref[...]
        s = s_ref[...]
        phit = phit_ref[...]
        cost = cost_ref[...]
        mf = jnp.maximum(m, phit)
        z = s * jnp.exp(m - mf) - jnp.exp(cost - mf) + jnp.exp(phit - mf)
        nll = mf + jnp.log(z) - phit
        out_ref[...] = (jnp.sum(nll) * (1.0 / N)).reshape(1, 1)


_tc_pass = pl.pallas_call(
    _tc_body,
    grid=(NB,),
    in_specs=[
        pl.BlockSpec((N, BC), lambda k: (0, k)),
        pl.BlockSpec((N, 1), lambda k: (0, 0)),
        pl.BlockSpec((N, 1), lambda k: (0, 0)),
    ],
    out_specs=pl.BlockSpec((1, 1), lambda k: (0, 0)),
    out_shape=jax.ShapeDtypeStruct((1, 1), jnp.float32),
    scratch_shapes=[
        pltpu.VMEM((N, 1), jnp.float32),
        pltpu.VMEM((N, 1), jnp.float32),
    ],
    compiler_params=pltpu.CompilerParams(
        dimension_semantics=("arbitrary",),
    ),
)


@jax.jit
def _impl(cos, phi, target):
    cos_t, phi_t = _sc_gather()(cos, phi, target)
    loss = _tc_pass(cos, cos_t.reshape(N, 1), phi_t.reshape(N, 1))
    return loss[0, 0]


def kernel(cos, phi, target):
    return _impl(cos, phi, target)


# speedup vs baseline: 2.1543x; 1.0010x over previous
import functools

import jax
import jax.numpy as jnp
from jax import lax
from jax.experimental import pallas as pl
from jax.experimental.pallas import tpu as pltpu
from jax.experimental.pallas import tpu_sc as plsc

N, C = 1024, 100000
BC = 2048
NB = (C + BC - 1) // BC
LAST = NB - 1

_NC, _NS, _L = 2, 16, 16
_NW = _NC * _NS
_BPW = N // _NW


def _sc_gather_body(cos_hbm, phi_hbm, tgt_hbm, cost_out, phit_out,
                    tgt_v, cwin, pwin, cg_v, pg_v, sem):
    wid = lax.axis_index("s") * _NC + lax.axis_index("c")
    base = wid * _BPW
    pltpu.sync_copy(tgt_hbm.at[pl.ds(base, _BPW)], tgt_v)
    copies = []
    for r in range(_BPW):
        if r % _L == 0:
            tgrp = tgt_v[pl.ds(r, _L)]
        t = tgrp[r % _L]
        c0 = pl.multiple_of(jnp.bitwise_and(t, -128), 128)
        r0 = (r // 8) * 8
        copies.append(pltpu.async_copy(
            cos_hbm.at[pl.ds(base + r0, 8), pl.ds(c0, 128)],
            cwin.at[pl.ds(r * 8, 8)], sem))
        copies.append(pltpu.async_copy(
            phi_hbm.at[pl.ds(base + r0, 8), pl.ds(c0, 128)],
            pwin.at[pl.ds(r * 8, 8)], sem))
    for cp in copies:
        cp.wait()
    for j in range(_BPW // _L):
        tv = tgt_v[pl.ds(j * _L, _L)]
        off = jnp.bitwise_and(tv, 127)
        r = j * _L + lax.iota(jnp.int32, _L)
        rows = r * 8 + jnp.bitwise_and(r, 7)
        cg_v[pl.ds(j * _L, _L)] = plsc.load_gather(cwin, [rows, off])
        pg_v[pl.ds(j * _L, _L)] = plsc.load_gather(pwin, [rows, off])
    pltpu.sync_copy(cg_v, cost_out.at[pl.ds(base, _BPW)])
    pltpu.sync_copy(pg_v, phit_out.at[pl.ds(base, _BPW)])


@functools.cache
def _sc_gather():
    return functools.partial(
        pl.kernel,
        out_type=(jax.ShapeDtypeStruct((N,), jnp.float32),
                  jax.ShapeDtypeStruct((N,), jnp.float32)),
        mesh=plsc.VectorSubcoreMesh(core_axis_name="c", subcore_axis_name="s",
                                    num_cores=_NC, num_subcores=_NS),
        scratch_types=[
            pltpu.VMEM((_BPW,), jnp.int32),
            pltpu.VMEM((_BPW * 8, 128), jnp.float32),
            pltpu.VMEM((_BPW * 8, 128), jnp.float32),
            pltpu.VMEM((_BPW,), jnp.float32),
            pltpu.VMEM((_BPW,), jnp.float32),
            pltpu.SemaphoreType.DMA,
        ],
        compiler_params=pltpu.CompilerParams(needs_layout_passes=False,
                                             use_tc_tiling_on_sc=True),
    )(_sc_gather_body)


def _tc_body(cos_ref, cost_ref, phit_ref, out_ref, m_ref, s_ref):
    k = pl.program_id(0)

    @pl.when(k == 0)
    def _init():
        m_ref[...] = jnp.full(m_ref.shape, -jnp.inf, jnp.float32)
        s_ref[...] = jnp.zeros(s_ref.shape, jnp.float32)

    def update(block):
        m = m_ref[...]
        bm = jnp.max(block, axis=1, keepdims=True)
        mn = jnp.maximum(m, bm)
        s_ref[...] = (s_ref[...] * jnp.exp(m - mn)
                      + jnp.sum(jnp.exp(block - mn), axis=1, keepdims=True))
        m_ref[...] = mn

    @pl.when(k < LAST)
    def _full():
        update(cos_ref[...])

    @pl.when(k == LAST)
    def _last():
        block = cos_ref[...]
        ids = lax.broadcasted_iota(jnp.int32, block.shape, 1) + k * BC
        update(jnp.where(ids < C, block, -jnp.inf))
        m = m_ref[...]
        s = s_ref[...]
        phit = phit_ref[...]
        cost = cost_ref[...]
        mf = jnp.maximum(m, phit)
        z = s * jnp.exp(m - mf) - jnp.exp(cost - mf) + jnp.exp(phit - mf)
        nll = mf + jnp.log(z) - phit
        out_ref[...] = (jnp.sum(nll) * (1.0 / N)).reshape(1, 1)


_tc_pass = pl.pallas_call(
    _tc_body,
    grid=(NB,),
    in_specs=[
        pl.BlockSpec((N, BC), lambda k: (0, k)),
        pl.BlockSpec((N, 1), lambda k: (0, 0)),
        pl.BlockSpec((N, 1), lambda k: (0, 0)),
    ],
    out_specs=pl.BlockSpec((1, 1), lambda k: (0, 0)),
    out_shape=jax.ShapeDtypeStruct((1, 1), jnp.float32),
    scratch_shapes=[
        pltpu.VMEM((N, 1), jnp.float32),
        pltpu.VMEM((N, 1), jnp.float32),
    ],
    compiler_params=pltpu.CompilerParams(
        dimension_semantics=("arbitrary",),
    ),
)


@jax.jit
def _impl(cos, phi, target):
    cos_t, phi_t = _sc_gather()(cos, phi, target)
    loss = _tc_pass(cos, cos_t.reshape(N, 1), phi_t.reshape(N, 1))
    return loss[0, 0]


def kernel(cos, phi, target):
    return _impl(cos, phi, target)


# speedup vs baseline: 2.1789x; 1.0114x over previous
import jax
import jax.numpy as jnp
from jax import lax
from jax.experimental import pallas as pl
from jax.experimental.pallas import tpu as pltpu

N, C = 1024, 100000
BC = 2048
NB = (C + BC - 1) // BC
LAST = NB - 1
ISSUE_STEPS = 32
RPS = N // ISSUE_STEPS


def _window_copies(tgt, cosw, phiw, cwin, pwin, sem_c, sem_p, i):
    t = tgt[i]
    c0 = pl.multiple_of(jnp.bitwise_and(t, -128), 128)
    cc = pltpu.make_async_copy(cosw.at[pl.ds(i, 1), pl.ds(c0, 128)],
                               cwin.at[pl.ds(i, 1)], sem_c)
    pc = pltpu.make_async_copy(phiw.at[pl.ds(i, 1), pl.ds(c0, 128)],
                               pwin.at[pl.ds(i, 1)], sem_p)
    return cc, pc


def _body(tgt, cos_ref, tvec_ref, cosw, phiw, out_ref,
          m_ref, s_ref, cwin, pwin, sem_c, sem_p):
    k = pl.program_id(0)

    @pl.when(k == 0)
    def _init():
        m_ref[...] = jnp.full(m_ref.shape, -jnp.inf, jnp.float32)
        s_ref[...] = jnp.zeros(s_ref.shape, jnp.float32)

    @pl.when(k < ISSUE_STEPS)
    def _issue():
        def body(i, carry):
            cc, pc = _window_copies(tgt, cosw, phiw, cwin, pwin,
                                    sem_c, sem_p, i)
            cc.start()
            pc.start()
            return carry
        lax.fori_loop(k * RPS, (k + 1) * RPS, body, 0)

    def update(block):
        m = m_ref[...]
        bm = jnp.max(block, axis=1, keepdims=True)
        mn = jnp.maximum(m, bm)
        s_ref[...] = (s_ref[...] * jnp.exp(m - mn)
                      + jnp.sum(jnp.exp(block - mn), axis=1, keepdims=True))
        m_ref[...] = mn

    @pl.when(k < LAST)
    def _full():
        update(cos_ref[...])

    @pl.when(k == LAST)
    def _last():
        block = cos_ref[...]
        ids = lax.broadcasted_iota(jnp.int32, block.shape, 1) + k * BC
        update(jnp.where(ids < C, block, -jnp.inf))

        def wbody(i, carry):
            cc, pc = _window_copies(tgt, cosw, phiw, cwin, pwin,
                                    sem_c, sem_p, i)
            cc.wait()
            pc.wait()
            return carry
        lax.fori_loop(0, N, wbody, 0)

        off = jnp.bitwise_and(tvec_ref[...], 127)
        lane = lax.broadcasted_iota(jnp.int32, (N, 128), 1)
        sel = lane == off
        cost = jnp.sum(jnp.where(sel, cwin[...], 0.0), axis=1, keepdims=True)
        phit = jnp.sum(jnp.where(sel, pwin[...], 0.0), axis=1, keepdims=True)

        m = m_ref[...]
        s = s_ref[...]
        mf = jnp.maximum(m, phit)
        z = s * jnp.exp(m - mf) - jnp.exp(cost - mf) + jnp.exp(phit - mf)
        nll = mf + jnp.log(z) - phit
        out_ref[...] = (jnp.sum(nll) * (1.0 / N)).reshape(1, 1)


_grid_spec = pltpu.PrefetchScalarGridSpec(
    num_scalar_prefetch=1,
    grid=(NB,),
    in_specs=[
        pl.BlockSpec((N, BC), lambda k, tgt: (0, k)),
        pl.BlockSpec((N, 1), lambda k, tgt: (0, 0)),
        pl.BlockSpec(memory_space=pl.ANY),
        pl.BlockSpec(memory_space=pl.ANY),
    ],
    out_specs=pl.BlockSpec((1, 1), lambda k, tgt: (0, 0)),
    scratch_shapes=[
        pltpu.VMEM((N, 1), jnp.float32),
        pltpu.VMEM((N, 1), jnp.float32),
        pltpu.VMEM((N, 128), jnp.float32),
        pltpu.VMEM((N, 128), jnp.float32),
        pltpu.SemaphoreType.DMA,
        pltpu.SemaphoreType.DMA,
    ],
)

_pass = pl.pallas_call(
    _body,
    grid_spec=_grid_spec,
    out_shape=jax.ShapeDtypeStruct((1, 1), jnp.float32),
    compiler_params=pltpu.CompilerParams(
        dimension_semantics=("arbitrary",),
    ),
)


@jax.jit
def _impl(cos, phi, target):
    loss = _pass(target, cos, target.reshape(N, 1), cos, phi)
    return loss[0, 0]


def kernel(cos, phi, target):
    return _impl(cos, phi, target)


# speedup vs baseline: 12.2710x; 5.6317x over previous
import jax
import jax.numpy as jnp
from jax import lax
from jax.experimental import pallas as pl
from jax.experimental.pallas import tpu as pltpu

N, C = 1024, 100000
BR = 2048
NB = (C + BR - 1) // BR
LAST = NB - 1
ISSUE_STEPS = 32
RPS = N // ISSUE_STEPS


def _window_copies(tgt, cosw, phiw, cwin, pwin, sem_c, sem_p, i):
    t = tgt[i]
    i0 = pl.multiple_of(jnp.bitwise_and(i, -128), 128)
    cc = pltpu.make_async_copy(cosw.at[pl.ds(t, 1), pl.ds(i0, 128)],
                               cwin.at[pl.ds(i, 1)], sem_c)
    pc = pltpu.make_async_copy(phiw.at[pl.ds(t, 1), pl.ds(i0, 128)],
                               pwin.at[pl.ds(i, 1)], sem_p)
    return cc, pc


def _body(tgt, cos_ref, cosw, phiw, out_ref,
          m_ref, s_ref, cwin, pwin, sem_c, sem_p):
    k = pl.program_id(0)

    @pl.when(k == 0)
    def _init():
        m_ref[...] = jnp.full(m_ref.shape, -jnp.inf, jnp.float32)
        s_ref[...] = jnp.zeros(s_ref.shape, jnp.float32)

    @pl.when(k < ISSUE_STEPS)
    def _issue():
        def body(i, carry):
            cc, pc = _window_copies(tgt, cosw, phiw, cwin, pwin,
                                    sem_c, sem_p, i)
            cc.start()
            pc.start()
            return carry
        lax.fori_loop(k * RPS, (k + 1) * RPS, body, 0)

    def update(block):
        m = m_ref[...]
        bm = jnp.max(block, axis=0, keepdims=True)
        mn = jnp.maximum(m, bm)
        s_ref[...] = (s_ref[...] * jnp.exp(m - mn)
                      + jnp.sum(jnp.exp(block - mn), axis=0, keepdims=True))
        m_ref[...] = mn

    @pl.when(k < LAST)
    def _full():
        update(cos_ref[...])

    @pl.when(k == LAST)
    def _last():
        block = cos_ref[...]
        ids = lax.broadcasted_iota(jnp.int32, block.shape, 0) + k * BR
        update(jnp.where(ids < C, block, -jnp.inf))

        def wbody(i, carry):
            cc, pc = _window_copies(tgt, cosw, phiw, cwin, pwin,
                                    sem_c, sem_p, i)
            cc.wait()
            pc.wait()
            return carry
        lax.fori_loop(0, N, wbody, 0)

        row = lax.broadcasted_iota(jnp.int32, (N, 128), 0)
        lanes = lax.broadcasted_iota(jnp.int32, (N, 128), 1)
        sel = lanes == jnp.bitwise_and(row, 127)
        cost = jnp.sum(jnp.where(sel, cwin[...], 0.0), axis=1, keepdims=True)
        phit = jnp.sum(jnp.where(sel, pwin[...], 0.0), axis=1, keepdims=True)
        cost = jnp.transpose(cost)
        phit = jnp.transpose(phit)

        m = m_ref[...]
        s = s_ref[...]
        mf = jnp.maximum(m, phit)
        z = s * jnp.exp(m - mf) - jnp.exp(cost - mf) + jnp.exp(phit - mf)
        nll = mf + jnp.log(z) - phit
        out_ref[...] = (jnp.sum(nll) * (1.0 / N)).reshape(1, 1)


_grid_spec = pltpu.PrefetchScalarGridSpec(
    num_scalar_prefetch=1,
    grid=(NB,),
    in_specs=[
        pl.BlockSpec((BR, N), lambda k, tgt: (k, 0)),
        pl.BlockSpec(memory_space=pl.ANY),
        pl.BlockSpec(memory_space=pl.ANY),
    ],
    out_specs=pl.BlockSpec((1, 1), lambda k, tgt: (0, 0)),
    scratch_shapes=[
        pltpu.VMEM((1, N), jnp.float32),
        pltpu.VMEM((1, N), jnp.float32),
        pltpu.VMEM((N, 128), jnp.float32),
        pltpu.VMEM((N, 128), jnp.float32),
        pltpu.SemaphoreType.DMA,
        pltpu.SemaphoreType.DMA,
    ],
)

_pass = pl.pallas_call(
    _body,
    grid_spec=_grid_spec,
    out_shape=jax.ShapeDtypeStruct((1, 1), jnp.float32),
    compiler_params=pltpu.CompilerParams(
        dimension_semantics=("arbitrary",),
    ),
)


@jax.jit
def _impl(cos, phi, target):
    cos_t = cos.T
    phi_t = phi.T
    loss = _pass(target, cos_t, cos_t, phi_t)
    return loss[0, 0]


def kernel(cos, phi, target):
    return _impl(cos, phi, target)
